# Initial kernel scaffold; baseline (speedup 1.0000x reference)
import jax
import jax.numpy as jnp
from jax.experimental import pallas as pl


def kernel(cate_indices, fm_table, lr_table, lr_bias):
    raise NotImplementedError("write your pallas kernel here")


# speedup vs baseline: 18.5964x; 18.5964x over previous
import functools

import jax
import jax.numpy as jnp
from jax import lax
from jax.experimental import pallas as pl
from jax.experimental.pallas import tpu as pltpu
from jax.experimental.pallas import tpu_sc as plsc

B = 16384
F = 26
V = 1000000
D = 16

NC = 2
NS = 16
NW = NC * NS
B_PER_W = B // NW
IDX_COLS = 104
IDX_ROWS = (B_PER_W * F) // IDX_COLS
CB = 32
ROWS_PER_CHUNK = CB * F
GROWS = ROWS_PER_CHUNK // IDX_COLS
NCHUNK = B_PER_W // CB

_mesh = plsc.VectorSubcoreMesh(core_axis_name="c", subcore_axis_name="s")


@functools.partial(
    pl.kernel,
    out_type=jax.ShapeDtypeStruct((B,), jnp.float32),
    mesh=_mesh,
    compiler_params=pltpu.CompilerParams(needs_layout_passes=False, use_tc_tiling_on_sc=False),
    scratch_types=[
        pltpu.VMEM((IDX_ROWS, IDX_COLS), jnp.int32),
        pltpu.VMEM((ROWS_PER_CHUNK, D), jnp.float32),
        pltpu.VMEM((ROWS_PER_CHUNK + 16,), jnp.float32),
        pltpu.VMEM((256,), jnp.float32),
        pltpu.VMEM((B_PER_W,), jnp.float32),
        pltpu.VMEM((16,), jnp.float32),
        pltpu.SemaphoreType.DMA,
        pltpu.SemaphoreType.DMA,
    ],
)
def _fm_sc(idx_hbm, fm_hbm, lr_hbm, bias_hbm, out_hbm,
           idx_v, rows_v, lr_v, tm_v, out_v, bias_v, sem_fm, sem_lr):
    wid = lax.axis_index("s") * NC + lax.axis_index("c")
    pltpu.sync_copy(idx_hbm.at[wid], idx_v)
    pltpu.sync_copy(bias_hbm, bias_v.at[pl.ds(0, 1)])
    bias_s = bias_v[pl.ds(0, 16)][0]
    lane = lax.iota(jnp.int32, 16)
    mask10 = lane < 10
    zero16 = jnp.zeros((16,), jnp.float32)

    def chunk_body(c, carry):
        copies = []
        for j in range(GROWS):
            row = c * GROWS + j
            copies.append(pltpu.async_copy(
                fm_hbm.at[idx_v.at[row]],
                rows_v.at[pl.ds(j * IDX_COLS, IDX_COLS)], sem_fm))
            copies.append(pltpu.async_copy(
                lr_hbm.at[idx_v.at[row]],
                lr_v.at[pl.ds(j * IDX_COLS, IDX_COLS)], sem_lr))
        for cp in copies:
            cp.wait()
        for g in range(CB // 16):
            for bb in range(16):
                b = g * 16 + bb
                s = zero16
                q = zero16
                for f in range(F):
                    e = rows_v[b * F + f]
                    s = s + e
                    q = q + e * e
                t = 0.5 * (s * s - q)
                l1 = lr_v[pl.ds(b * F, 16)]
                l2 = jnp.where(mask10, lr_v[pl.ds(b * F + 16, 16)], 0.0)
                tm_v[pl.ds(bb * 16, 16)] = t + l1 + l2
            acc = jnp.full((16,), bias_s, jnp.float32)
            for dcol in range(16):
                acc = acc + plsc.load_gather(tm_v, [lane * 16 + dcol])
            out_v[pl.ds(c * CB + g * 16, 16)] = acc
        return carry

    lax.fori_loop(0, NCHUNK, chunk_body, 0)
    pltpu.sync_copy(out_v, out_hbm.at[pl.ds(wid * B_PER_W, B_PER_W)])


def kernel(cate_indices, fm_table, lr_table, lr_bias):
    idx = cate_indices.astype(jnp.int32).reshape(NW, IDX_ROWS, IDX_COLS)
    lr_flat = lr_table.reshape(V)
    out = _fm_sc(idx, fm_table, lr_flat, lr_bias)
    return out.reshape(B, 1)
